# baseline (speedup 1.0000x reference)
import functools

import jax
import jax.numpy as jnp
from jax import lax
from jax.experimental import pallas as pl
from jax.experimental.pallas import tpu as pltpu
from jax.experimental.pallas import tpu_sc as plsc

_N_CTX = 4
_SEQ = 77
_DIM = 512
_MAIN = 72
_TAIL = _SEQ - _MAIN
_SRW = 8
_SRLEN = _SRW * _MAIN + _SRW * _TAIL + 8
_LANES = 16


def _sc_prompt_gather(idx_flat, table, ctx_flat, n_cls):
    info = plsc.get_sparse_core_info()
    nw = info.num_cores * info.num_subcores
    wlen = idx_flat.shape[0] // nw
    nsr = wlen // _SRLEN
    nfull = n_cls // nw
    rem = n_cls % nw
    mesh = plsc.VectorSubcoreMesh(core_axis_name="c", subcore_axis_name="s")

    @functools.partial(
        pl.kernel,
        mesh=mesh,
        out_type=jax.ShapeDtypeStruct((n_cls, _SEQ, _DIM), jnp.float32),
        scratch_types=[
            pltpu.VMEM((wlen,), jnp.int32),
            pltpu.VMEM((_N_CTX * _DIM,), jnp.float32),
            pltpu.VMEM((_SEQ, _DIM), jnp.float32),
            pltpu.VMEM((_SEQ, _DIM), jnp.float32),
            pltpu.VMEM((_SRW * _TAIL, _DIM), jnp.float32),
            pltpu.SemaphoreType.DMA,
            pltpu.SemaphoreType.DMA,
            pltpu.SemaphoreType.DMA,
            pltpu.SemaphoreType.DMA,
            pltpu.SemaphoreType.DMA,
        ],
    )
    def k(idx_hbm, table_hbm, ctx_hbm, out_hbm,
          idx_v, ctx_v, bufa, bufb, pool, gsa, gsb, gst, wsa, wsb):
        wid = lax.axis_index("s") * info.num_cores + lax.axis_index("c")
        nr = nfull + (wid < rem).astype(jnp.int32)

        pltpu.sync_copy(idx_hbm.at[pl.ds(wid * wlen, wlen)], idx_v)
        pltpu.sync_copy(ctx_hbm, ctx_v)

        def issue_main(s, j, buf, sem):
            pltpu.async_copy(
                table_hbm.at[idx_v.at[pl.ds(s * _SRLEN + j * _MAIN, _MAIN)]],
                buf.at[pl.ds(0, _MAIN)], sem)

        def wait_main(buf, sem):
            pltpu.make_async_copy(
                table_hbm.at[idx_v.at[pl.ds(0, _MAIN)]],
                buf.at[pl.ds(0, _MAIN)], sem).wait()

        def issue_tail(s):
            pltpu.async_copy(
                table_hbm.at[idx_v.at[pl.ds(s * _SRLEN + _SRW * _MAIN,
                                            _SRW * _TAIL)]], pool, gst)

        def wait_tail():
            pltpu.make_async_copy(
                table_hbm.at[idx_v.at[pl.ds(0, _SRW * _TAIL)]],
                pool, gst).wait()

        def wait_write(buf, ws):
            pltpu.make_async_copy(buf, out_hbm.at[0], ws).wait()

        issue_tail(0)
        issue_main(0, 0, bufa, gsa)

        def body(s, _):
            wait_tail()
            for j in range(_SRW):
                buf, sem, ws = ((bufa, gsa, wsa) if j % 2 == 0
                                else (bufb, gsb, wsb))
                obuf, osem, ows = ((bufb, gsb, wsb) if j % 2 == 0
                                   else (bufa, gsa, wsa))
                r = s * _SRW + j

                @pl.when(r < nr)
                def _(j=j, r=r, buf=buf, sem=sem, ws=ws,
                      obuf=obuf, osem=osem, ows=ows):
                    wait_main(buf, sem)
                    for t in range(_TAIL):
                        for i in range(_DIM // _LANES):
                            buf[_MAIN + t, pl.ds(i * _LANES, _LANES)] = (
                                pool[j * _TAIL + t,
                                     pl.ds(i * _LANES, _LANES)])
                    if j == _SRW - 1:
                        @pl.when(s + 1 < nsr)
                        def _():
                            issue_tail(s + 1)

                    nj = (j + 1) % _SRW
                    ns = s + (1 if j == _SRW - 1 else 0)

                    @pl.when(r + 1 < nr)
                    def _():
                        @pl.when(r >= 1)
                        def _():
                            wait_write(obuf, ows)

                        issue_main(ns, nj, obuf, osem)

                    for t in range(_N_CTX):
                        for i in range(_DIM // _LANES):
                            buf[1 + t, pl.ds(i * _LANES, _LANES)] = (
                                ctx_v[pl.ds(t * _DIM + i * _LANES, _LANES)])

                    c = r * nw + wid
                    pltpu.async_copy(buf, out_hbm.at[c], ws)

            return _

        lax.fori_loop(0, nsr, body, None)
        wait_write(bufa, wsa)
        wait_write(bufb, wsb)

    return k(idx_flat, table, ctx_flat)


def kernel(tokenized_prompts, token_embedding, ctx):
    n_cls = tokenized_prompts.shape[0]
    info = plsc.get_sparse_core_info()
    nw = info.num_cores * info.num_subcores
    rpw = -(-n_cls // (nw * _SRW)) * _SRW
    pad = nw * rpw - n_cls

    tokp = jnp.concatenate(
        [tokenized_prompts,
         jnp.zeros((pad, tokenized_prompts.shape[1]), jnp.int32)], axis=0)
    by_worker = tokp.reshape(rpw, nw, _SEQ).transpose(1, 0, 2)
    nsr = rpw // _SRW
    mains = by_worker[:, :, :_MAIN].reshape(nw, nsr, _SRW * _MAIN)
    tails = by_worker[:, :, _MAIN:].reshape(nw, nsr, _SRW * _TAIL)
    padb = jnp.zeros((nw, nsr, _SRLEN - _SRW * (_MAIN + _TAIL)), jnp.int32)
    idx_flat = jnp.concatenate([mains, tails, padb], axis=2).reshape(-1)

    return _sc_prompt_gather(idx_flat, token_embedding, ctx.reshape(-1),
                             n_cls)


# speedup vs baseline: 1.0043x; 1.0043x over previous
import functools

import jax
import jax.numpy as jnp
from jax import lax
from jax.experimental import pallas as pl
from jax.experimental.pallas import tpu as pltpu
from jax.experimental.pallas import tpu_sc as plsc

_N_CTX = 4
_SEQ = 77
_DIM = 512
_MAIN = 72
_TAIL = _SEQ - _MAIN
_SRW = 8
_SRLEN = _SRW * _MAIN + _SRW * _TAIL + 8
_LANES = 16


def _sc_prompt_gather(idx_flat, table, ctx_flat, n_cls):
    info = plsc.get_sparse_core_info()
    nw = info.num_cores * info.num_subcores
    wlen = idx_flat.shape[0] // nw
    nsr = wlen // _SRLEN
    nfull = n_cls // nw
    rem = n_cls % nw
    mesh = plsc.VectorSubcoreMesh(core_axis_name="c", subcore_axis_name="s")

    @functools.partial(
        pl.kernel,
        mesh=mesh,
        out_type=jax.ShapeDtypeStruct((n_cls, _SEQ, _DIM), jnp.float32),
        scratch_types=[
            pltpu.VMEM((wlen,), jnp.int32),
            pltpu.VMEM((_N_CTX * _DIM,), jnp.float32),
            pltpu.VMEM((_SEQ, _DIM), jnp.float32),
            pltpu.VMEM((_SEQ, _DIM), jnp.float32),
            pltpu.VMEM((_SRW * _TAIL, _DIM), jnp.float32),
            pltpu.SemaphoreType.DMA,
            pltpu.SemaphoreType.DMA,
            pltpu.SemaphoreType.DMA,
        ],
    )
    def k(idx_hbm, table_hbm, ctx_hbm, out_hbm,
          idx_v, ctx_v, bufa, bufb, pool, gsa, gsb, gst):
        wid = lax.axis_index("s") * info.num_cores + lax.axis_index("c")
        nr = nfull + (wid < rem).astype(jnp.int32)

        pltpu.sync_copy(idx_hbm.at[pl.ds(wid * wlen, wlen)], idx_v)
        pltpu.sync_copy(ctx_hbm, ctx_v)

        def issue_main(s, j, buf, sem):
            pltpu.async_copy(
                table_hbm.at[idx_v.at[pl.ds(s * _SRLEN + j * _MAIN, _MAIN)]],
                buf.at[pl.ds(0, _MAIN)], sem)

        def wait_main(buf, sem):
            pltpu.make_async_copy(
                table_hbm.at[idx_v.at[pl.ds(0, _MAIN)]],
                buf.at[pl.ds(0, _MAIN)], sem).wait()

        def issue_tail(s):
            pltpu.async_copy(
                table_hbm.at[idx_v.at[pl.ds(s * _SRLEN + _SRW * _MAIN,
                                            _SRW * _TAIL)]], pool, gst)

        def wait_tail():
            pltpu.make_async_copy(
                table_hbm.at[idx_v.at[pl.ds(0, _SRW * _TAIL)]],
                pool, gst).wait()

        issue_tail(0)
        issue_main(0, 0, bufa, gsa)

        def body(s, _):
            wait_tail()
            for j in range(_SRW):
                buf, sem = (bufa, gsa) if j % 2 == 0 else (bufb, gsb)
                obuf, osem = (bufb, gsb) if j % 2 == 0 else (bufa, gsa)
                r = s * _SRW + j

                @pl.when(r < nr)
                def _(j=j, r=r, buf=buf, sem=sem, obuf=obuf, osem=osem):
                    wait_main(buf, sem)
                    for t in range(_TAIL):
                        for i in range(_DIM // _LANES):
                            buf[_MAIN + t, pl.ds(i * _LANES, _LANES)] = (
                                pool[j * _TAIL + t,
                                     pl.ds(i * _LANES, _LANES)])
                    if j == _SRW - 1:
                        @pl.when(s + 1 < nsr)
                        def _():
                            issue_tail(s + 1)

                    nj = (j + 1) % _SRW
                    ns = s + (1 if j == _SRW - 1 else 0)

                    @pl.when(r + 1 < nr)
                    def _():
                        issue_main(ns, nj, obuf, osem)

                    for t in range(_N_CTX):
                        for i in range(_DIM // _LANES):
                            buf[1 + t, pl.ds(i * _LANES, _LANES)] = (
                                ctx_v[pl.ds(t * _DIM + i * _LANES, _LANES)])

                    c = r * nw + wid
                    pltpu.sync_copy(buf, out_hbm.at[c])

            return _

        lax.fori_loop(0, nsr, body, None)

    return k(idx_flat, table, ctx_flat)


def kernel(tokenized_prompts, token_embedding, ctx):
    n_cls = tokenized_prompts.shape[0]
    info = plsc.get_sparse_core_info()
    nw = info.num_cores * info.num_subcores
    rpw = -(-n_cls // (nw * _SRW)) * _SRW
    pad = nw * rpw - n_cls

    tokp = jnp.concatenate(
        [tokenized_prompts,
         jnp.zeros((pad, tokenized_prompts.shape[1]), jnp.int32)], axis=0)
    by_worker = tokp.reshape(rpw, nw, _SEQ).transpose(1, 0, 2)
    nsr = rpw // _SRW
    mains = by_worker[:, :, :_MAIN].reshape(nw, nsr, _SRW * _MAIN)
    tails = by_worker[:, :, _MAIN:].reshape(nw, nsr, _SRW * _TAIL)
    padb = jnp.zeros((nw, nsr, _SRLEN - _SRW * (_MAIN + _TAIL)), jnp.int32)
    idx_flat = jnp.concatenate([mains, tails, padb], axis=2).reshape(-1)

    return _sc_prompt_gather(idx_flat, token_embedding, ctx.reshape(-1),
                             n_cls)
